# Initial kernel scaffold; baseline (speedup 1.0000x reference)
import functools

import jax
import jax.numpy as jnp
from jax import lax
from jax.experimental import pallas as pl
from jax.experimental.pallas import tpu as pltpu
from jax.experimental.pallas import tpu_sc as plsc

B = 16384
K = 64
NNEG = 20
NC = 2
NS = 16
L = 16
NW = NC * NS
BW = B // NW
CE = 64
NCH = BW // CE
IDX_CHUNK = 128


def _body(user_hbm, pos_hbm, neg_hbm, eu_hbm, ei_hbm, out_hbm,
          uidx_v, pidx_v, nidx_v, urows_v, prows_v, nrows_v,
          posout_v, negout_v, sem):
    c = lax.axis_index("c")
    s = lax.axis_index("s")
    wid = s * NC + c
    base = wid * BW

    lanes = lax.broadcasted_iota(jnp.int32, (L,), 0)

    def chunk_body(ch, carry):
        eb = base + ch * CE
        pltpu.sync_copy(user_hbm.at[pl.ds(eb, CE)], uidx_v)
        pltpu.sync_copy(pos_hbm.at[pl.ds(eb, CE)], pidx_v)
        pltpu.sync_copy(neg_hbm.at[pl.ds(eb * NNEG, CE * NNEG)], nidx_v)
        cps = [
            pltpu.async_copy(eu_hbm.at[uidx_v], urows_v, sem),
            pltpu.async_copy(ei_hbm.at[pidx_v], prows_v, sem),
        ]
        for q in range(CE * NNEG // IDX_CHUNK):
            cps.append(pltpu.async_copy(
                ei_hbm.at[nidx_v.at[pl.ds(q * IDX_CHUNK, IDX_CHUNK)]],
                nrows_v.at[pl.ds(q * IDX_CHUNK, IDX_CHUNK)],
                sem))
        for cp in cps:
            cp.wait()

        for g in range(CE // L):
            e16 = lanes + g * L
            ubase = e16 * K
            nbase = e16 * (NNEG * K)

            for blk, jlist in ((0, tuple(range(0, 10))),
                               (1, tuple(range(10, 20)))):
                njs = len(jlist)

                def kbody(k, kcarry, blk=blk, jlist=jlist):
                    pacc = kcarry[0]
                    accs = list(kcarry[1:])
                    u = plsc.load_gather(urows_v, [ubase + k])
                    if blk == 0:
                        p = plsc.load_gather(prows_v, [ubase + k])
                        pacc = pacc + u * p
                    for t, j in enumerate(jlist):
                        n = plsc.load_gather(nrows_v, [nbase + (j * K + k)])
                        accs[t] = accs[t] + u * n
                    return (pacc, *accs)

                zero = jnp.zeros((L,), jnp.float32)
                outc = lax.fori_loop(0, K, kbody, (zero,) * (njs + 1))
                if blk == 0:
                    posout_v[pl.ds(g * L, L)] = outc[0]
                for t, j in enumerate(jlist):
                    plsc.store_scatter(negout_v, [e16 * NNEG + j], -outc[1 + t])

        pltpu.sync_copy(posout_v, out_hbm.at[pl.ds(eb, CE)])
        pltpu.sync_copy(negout_v, out_hbm.at[pl.ds(B + eb * NNEG, CE * NNEG)])
        return carry

    lax.fori_loop(0, NCH, chunk_body, 0)


_mesh = plsc.VectorSubcoreMesh(core_axis_name="c", subcore_axis_name="s")

_svd = functools.partial(
    pl.kernel,
    mesh=_mesh,
    out_type=jax.ShapeDtypeStruct((B + B * NNEG,), jnp.float32),
    scratch_types=[
        pltpu.VMEM((CE,), jnp.int32),
        pltpu.VMEM((CE,), jnp.int32),
        pltpu.VMEM((CE * NNEG,), jnp.int32),
        pltpu.VMEM((CE * K,), jnp.float32),
        pltpu.VMEM((CE * K,), jnp.float32),
        pltpu.VMEM((CE * NNEG * K,), jnp.float32),
        pltpu.VMEM((CE,), jnp.float32),
        pltpu.VMEM((CE * NNEG,), jnp.float32),
        pltpu.SemaphoreType.DMA,
    ],
)(_body)


def kernel(user, pos_item, neg_item, embedding_user, embedding_item):
    user = user.astype(jnp.int32)
    pos = pos_item.astype(jnp.int32)
    neg = neg_item.astype(jnp.int32).reshape(-1)
    return _svd(user, pos, neg, embedding_user, embedding_item)


# speedup vs baseline: 4.0219x; 4.0219x over previous
import functools

import jax
import jax.numpy as jnp
from jax import lax
from jax.experimental import pallas as pl
from jax.experimental.pallas import tpu as pltpu
from jax.experimental.pallas import tpu_sc as plsc

B = 16384
K = 64
NNEG = 20
NC = 2
NS = 16
L = 16
NW = NC * NS
BW = B // NW
CE = 64
NCH = BW // CE
IDX_CHUNK = 128


def _body(user_hbm, pos_hbm, neg_hbm, eu_hbm, ei_hbm, out_hbm,
          uidx_v, pidx_v, nidx_v, urows_v, prows_v, nrows_v,
          posout_v, negout_v, sem):
    c = lax.axis_index("c")
    s = lax.axis_index("s")
    wid = s * NC + c
    base = wid * BW

    lanes = lax.broadcasted_iota(jnp.int32, (L,), 0)

    def chunk_body(ch, carry):
        eb = base + ch * CE
        pltpu.sync_copy(user_hbm.at[pl.ds(eb, CE)], uidx_v)
        pltpu.sync_copy(pos_hbm.at[pl.ds(eb, CE)], pidx_v)
        pltpu.sync_copy(neg_hbm.at[pl.ds(eb * NNEG, CE * NNEG)], nidx_v)
        cps = [
            pltpu.async_copy(eu_hbm.at[uidx_v], urows_v, sem),
            pltpu.async_copy(ei_hbm.at[pidx_v], prows_v, sem),
        ]
        for q in range(CE * NNEG // IDX_CHUNK):
            cps.append(pltpu.async_copy(
                ei_hbm.at[nidx_v.at[pl.ds(q * IDX_CHUNK, IDX_CHUNK)]],
                nrows_v.at[pl.ds(q * IDX_CHUNK, IDX_CHUNK)],
                sem))
        for cp in cps:
            cp.wait()

        for g in range(CE // L):
            e16 = lanes + g * L
            nrow = e16 * NNEG

            for blk, jlist in ((0, tuple(range(0, 10))),
                               (1, tuple(range(10, 20)))):
                njs = len(jlist)

                def kbody(k, kcarry, blk=blk, jlist=jlist):
                    pacc = kcarry[0]
                    accs = list(kcarry[1:])
                    kv = lanes * 0 + k
                    u = plsc.load_gather(urows_v, [e16, kv])
                    if blk == 0:
                        p = plsc.load_gather(prows_v, [e16, kv])
                        pacc = pacc + u * p
                    for t, j in enumerate(jlist):
                        n = plsc.load_gather(nrows_v, [nrow + j, kv])
                        accs[t] = accs[t] + u * n
                    return (pacc, *accs)

                zero = jnp.zeros((L,), jnp.float32)
                outc = lax.fori_loop(0, K, kbody, (zero,) * (njs + 1))
                if blk == 0:
                    posout_v[pl.ds(g * L, L)] = outc[0]
                for t, j in enumerate(jlist):
                    plsc.store_scatter(negout_v, [e16 * NNEG + j], -outc[1 + t])

        pltpu.sync_copy(posout_v, out_hbm.at[pl.ds(eb, CE)])
        pltpu.sync_copy(negout_v, out_hbm.at[pl.ds(B + eb * NNEG, CE * NNEG)])
        return carry

    lax.fori_loop(0, NCH, chunk_body, 0)


_mesh = plsc.VectorSubcoreMesh(core_axis_name="c", subcore_axis_name="s")

_svd = functools.partial(
    pl.kernel,
    mesh=_mesh,
    compiler_params=pltpu.CompilerParams(needs_layout_passes=False,
                                         use_tc_tiling_on_sc=False),
    out_type=jax.ShapeDtypeStruct((B + B * NNEG,), jnp.float32),
    scratch_types=[
        pltpu.VMEM((CE,), jnp.int32),
        pltpu.VMEM((CE,), jnp.int32),
        pltpu.VMEM((CE * NNEG,), jnp.int32),
        pltpu.VMEM((CE, K), jnp.float32),
        pltpu.VMEM((CE, K), jnp.float32),
        pltpu.VMEM((CE * NNEG, K), jnp.float32),
        pltpu.VMEM((CE,), jnp.float32),
        pltpu.VMEM((CE * NNEG,), jnp.float32),
        pltpu.SemaphoreType.DMA,
    ],
)(_body)


def kernel(user, pos_item, neg_item, embedding_user, embedding_item):
    user = user.astype(jnp.int32)
    pos = pos_item.astype(jnp.int32)
    neg = neg_item.astype(jnp.int32).reshape(-1)
    return _svd(user, pos, neg, embedding_user, embedding_item)


# speedup vs baseline: 4.1134x; 1.0227x over previous
import functools

import jax
import jax.numpy as jnp
from jax import lax
from jax.experimental import pallas as pl
from jax.experimental.pallas import tpu as pltpu
from jax.experimental.pallas import tpu_sc as plsc

B = 16384
K = 64
NNEG = 20
NC = 2
NS = 16
L = 16
NW = NC * NS
BW = B // NW
CE = 32
NCH = BW // CE
IDX_CHUNK = 128


def _body(user_hbm, pos_hbm, neg_hbm, eu_hbm, ei_hbm, out_hbm,
          uidx_v, pidx_v, nidx_v,
          urows0, prows0, nrows0, urows1, prows1, nrows1,
          posout_v, negout_v, sem0, sem1, osem):
    c = lax.axis_index("c")
    s = lax.axis_index("s")
    wid = s * NC + c
    base = wid * BW

    lanes = lax.broadcasted_iota(jnp.int32, (L,), 0)
    bufs = ((urows0, prows0, nrows0, sem0), (urows1, prows1, nrows1, sem1))

    pltpu.sync_copy(user_hbm.at[pl.ds(base, BW)], uidx_v)
    pltpu.sync_copy(pos_hbm.at[pl.ds(base, BW)], pidx_v)
    pltpu.sync_copy(neg_hbm.at[pl.ds(base * NNEG, BW * NNEG)], nidx_v)

    def fire(ch, buf):
        urows, prows, nrows, sem = buf
        cps = [
            pltpu.async_copy(eu_hbm.at[uidx_v.at[pl.ds(ch * CE, CE)]],
                             urows, sem),
            pltpu.async_copy(ei_hbm.at[pidx_v.at[pl.ds(ch * CE, CE)]],
                             prows, sem),
        ]
        for q in range(CE * NNEG // IDX_CHUNK):
            cps.append(pltpu.async_copy(
                ei_hbm.at[nidx_v.at[pl.ds(ch * CE * NNEG + q * IDX_CHUNK,
                                          IDX_CHUNK)]],
                nrows.at[pl.ds(q * IDX_CHUNK, IDX_CHUNK)],
                sem))
        return cps

    def compute(ch, buf):
        urows, prows, nrows, _ = buf
        for g in range(CE // L):
            e16 = lanes + g * L
            nrow = e16 * NNEG

            for blk, jlist in ((0, tuple(range(0, 10))),
                               (1, tuple(range(10, 20)))):
                njs = len(jlist)

                def kbody(k, kcarry, blk=blk, jlist=jlist):
                    pacc = kcarry[0]
                    accs = list(kcarry[1:])
                    kv = lanes * 0 + k
                    u = plsc.load_gather(urows, [e16, kv])
                    if blk == 0:
                        p = plsc.load_gather(prows, [e16, kv])
                        pacc = pacc + u * p
                    for t, j in enumerate(jlist):
                        n = plsc.load_gather(nrows, [nrow + j, kv])
                        accs[t] = accs[t] + u * n
                    return (pacc, *accs)

                zero = jnp.zeros((L,), jnp.float32)
                outc = lax.fori_loop(0, K, kbody, (zero,) * (njs + 1))
                eg16 = ch * CE + g * L + lanes
                if blk == 0:
                    posout_v[pl.ds(ch * CE + g * L, L)] = outc[0]
                for t, j in enumerate(jlist):
                    plsc.store_scatter(negout_v, [eg16 * NNEG + j],
                                       -outc[1 + t])

    inflight = fire(0, bufs[0])
    for ch in range(NCH):
        if ch + 1 < NCH:
            nxt = fire(ch + 1, bufs[(ch + 1) % 2])
        for cp in inflight:
            cp.wait()
        compute(ch, bufs[ch % 2])
        if ch + 1 < NCH:
            inflight = nxt

    o1 = pltpu.async_copy(posout_v, out_hbm.at[pl.ds(base, BW)], osem)
    o2 = pltpu.async_copy(negout_v,
                          out_hbm.at[pl.ds(B + base * NNEG, BW * NNEG)], osem)
    o1.wait()
    o2.wait()


_mesh = plsc.VectorSubcoreMesh(core_axis_name="c", subcore_axis_name="s")

_svd = functools.partial(
    pl.kernel,
    mesh=_mesh,
    compiler_params=pltpu.CompilerParams(needs_layout_passes=False,
                                         use_tc_tiling_on_sc=False),
    out_type=jax.ShapeDtypeStruct((B + B * NNEG,), jnp.float32),
    scratch_types=[
        pltpu.VMEM((BW,), jnp.int32),
        pltpu.VMEM((BW,), jnp.int32),
        pltpu.VMEM((BW * NNEG,), jnp.int32),
        pltpu.VMEM((CE, K), jnp.float32),
        pltpu.VMEM((CE, K), jnp.float32),
        pltpu.VMEM((CE * NNEG, K), jnp.float32),
        pltpu.VMEM((CE, K), jnp.float32),
        pltpu.VMEM((CE, K), jnp.float32),
        pltpu.VMEM((CE * NNEG, K), jnp.float32),
        pltpu.VMEM((BW,), jnp.float32),
        pltpu.VMEM((BW * NNEG,), jnp.float32),
        pltpu.SemaphoreType.DMA,
        pltpu.SemaphoreType.DMA,
        pltpu.SemaphoreType.DMA,
    ],
)(_body)


def kernel(user, pos_item, neg_item, embedding_user, embedding_item):
    user = user.astype(jnp.int32)
    pos = pos_item.astype(jnp.int32)
    neg = neg_item.astype(jnp.int32).reshape(-1)
    return _svd(user, pos, neg, embedding_user, embedding_item)
